# baseline (speedup 1.0000x reference)
import functools

import jax
import jax.numpy as jnp
from jax import lax
from jax.experimental import pallas as pl
from jax.experimental.pallas import tpu as pltpu

H = 32
D = 128
HID = 4096
SEQ_PREV = 8191
PAGE = 16
BUDGET = 2048
ROPE_THETA = 10000.0
P = (SEQ_PREV + 1) // PAGE
NSEL = BUDGET // PAGE
SBLK = 512
NSB = (SEQ_PREV + 1) // SBLK

INTERPRET = False


def _bf16_dot(a, b, dims):
    return lax.dot_general(a, b, dims, preferred_element_type=jnp.float32)


def _proj_kernel_rope(x_ref, w_ref, cos_ref, sin_ref, o_ref):
    t = _bf16_dot(x_ref[...], w_ref[...], (((1,), (1,)), ((), ())))
    c = cos_ref[...]
    s = sin_ref[...]
    x1 = t[:, : D // 2]
    x2 = t[:, D // 2:]
    o_ref[0] = jnp.concatenate([x1 * c - x2 * s, x2 * c + x1 * s], axis=1)


def _proj_kernel_plain(x_ref, w_ref, o_ref):
    o_ref[0] = _bf16_dot(x_ref[...], w_ref[...], (((1,), (1,)), ((), ())))


def _proj(x, w, cos=None, sin=None):
    rope = cos is not None
    in_specs = [
        pl.BlockSpec((1, HID), lambda i: (0, 0)),
        pl.BlockSpec((D, HID), lambda i: (i, 0)),
    ]
    args = [x, w]
    if rope:
        in_specs += [pl.BlockSpec((1, D // 2), lambda i: (0, 0)),
                     pl.BlockSpec((1, D // 2), lambda i: (0, 0))]
        args += [cos, sin]
    return pl.pallas_call(
        _proj_kernel_rope if rope else _proj_kernel_plain,
        grid=(HID // D,),
        in_specs=in_specs,
        out_specs=pl.BlockSpec((1, 1, D), lambda i: (i, 0, 0)),
        out_shape=jax.ShapeDtypeStruct((HID // D, 1, D), jnp.float32),
        interpret=INTERPRET,
    )(*args)


def _est_kernel(k2_ref, knew_ref, q_ref, est_ref):
    sb = pl.program_id(1)
    blk = k2_ref[...]
    row = sb * SBLK + lax.broadcasted_iota(jnp.int32, (SBLK, 1), 0)
    blk = jnp.where(row == SEQ_PREV, knew_ref[0], blk)
    pages = blk.reshape(SBLK // PAGE, PAGE, D)
    kmin = pages.min(axis=1)
    kmax = pages.max(axis=1)
    qh = q_ref[0]
    m = jnp.maximum(qh * kmin, qh * kmax)
    est_ref[0] = jnp.sum(m, axis=1, keepdims=True)


def _estimate(k2, knew, q):
    return pl.pallas_call(
        _est_kernel,
        grid=(H, NSB),
        in_specs=[
            pl.BlockSpec((SBLK, D), lambda h, sb: (sb, h)),
            pl.BlockSpec((1, 1, D), lambda h, sb: (h, 0, 0)),
            pl.BlockSpec((1, 1, D), lambda h, sb: (h, 0, 0)),
        ],
        out_specs=pl.BlockSpec((1, SBLK // PAGE, 1), lambda h, sb: (h, sb, 0)),
        out_shape=jax.ShapeDtypeStruct((H, P, 1), jnp.float32),
        compiler_params=pltpu.CompilerParams(
            dimension_semantics=("arbitrary", "arbitrary")),
        interpret=INTERPRET,
    )(k2, knew, q)


def _topk_pages(est):
    _, idx = lax.top_k(est, NSEL)
    return idx.astype(jnp.int32)


def _attn_kernel(idx_ref, k_ref, v_ref, q_ref, knew_ref, vnew_ref, o_ref,
                 acc_ref, m_ref, l_ref):
    h = pl.program_id(0)
    n = pl.program_id(1)

    @pl.when(n == 0)
    def _init():
        m_ref[0] = -jnp.inf
        l_ref[0] = 0.0
        acc_ref[...] = jnp.zeros_like(acc_ref)

    pid = idx_ref[h, n]
    kblk = k_ref[...]
    vblk = v_ref[...]
    ri = lax.broadcasted_iota(jnp.int32, (PAGE, 1), 0)
    last = jnp.logical_and(pid == P - 1, ri == PAGE - 1)
    kblk = jnp.where(last, knew_ref[0], kblk)
    vblk = jnp.where(last, vnew_ref[0], vblk)

    qh = q_ref[0]
    s = _bf16_dot(kblk, qh, (((1,), (1,)), ((), ())))
    s = s * (1.0 / (D ** 0.5))
    m_prev = m_ref[0]
    m_new = jnp.maximum(m_prev, jnp.max(s))
    corr = jnp.exp(m_prev - m_new)
    p = jnp.exp(s - m_new)
    l_ref[0] = l_ref[0] * corr + jnp.sum(p)
    pv = _bf16_dot(p, vblk, (((0,), (0,)), ((), ())))
    acc_ref[...] = acc_ref[...] * corr + pv
    m_ref[0] = m_new

    @pl.when(n == NSEL - 1)
    def _fin():
        o_ref[0] = acc_ref[...] / l_ref[0]


def _sparse_attn(k2, v2, q, knew, vnew, idx):
    grid_spec = pltpu.PrefetchScalarGridSpec(
        num_scalar_prefetch=1,
        grid=(H, NSEL),
        in_specs=[
            pl.BlockSpec((PAGE, D), lambda h, n, idx: (idx[h, n], h)),
            pl.BlockSpec((PAGE, D), lambda h, n, idx: (idx[h, n], h)),
            pl.BlockSpec((1, 1, D), lambda h, n, idx: (h, 0, 0)),
            pl.BlockSpec((1, 1, D), lambda h, n, idx: (h, 0, 0)),
            pl.BlockSpec((1, 1, D), lambda h, n, idx: (h, 0, 0)),
        ],
        out_specs=pl.BlockSpec((1, 1, D), lambda h, n, idx: (h, 0, 0)),
        scratch_shapes=[
            pltpu.VMEM((1, D), jnp.float32),
            pltpu.SMEM((1,), jnp.float32),
            pltpu.SMEM((1,), jnp.float32),
        ],
    )
    return pl.pallas_call(
        _attn_kernel,
        grid_spec=grid_spec,
        out_shape=jax.ShapeDtypeStruct((H, 1, D), jnp.float32),
        compiler_params=pltpu.CompilerParams(
            dimension_semantics=("arbitrary", "arbitrary")),
        interpret=INTERPRET,
    )(idx, k2, v2, q, knew, vnew)


def kernel(hidden_states, k_cache, v_cache, Wq, Wk, Wv, Wo):
    x = hidden_states.reshape(1, HID)
    d2 = D // 2
    inv_freq = 1.0 / (ROPE_THETA ** (jnp.arange(0, d2, dtype=jnp.float32) / d2))
    ang = jnp.float32(SEQ_PREV) * inv_freq
    cos = jnp.cos(ang).reshape(1, d2)
    sin = jnp.sin(ang).reshape(1, d2)

    q = _proj(x, Wq, cos, sin)
    knew = _proj(x, Wk, cos, sin)
    vnew = _proj(x, Wv)

    k2 = k_cache.reshape(SEQ_PREV, HID)
    v2 = v_cache.reshape(SEQ_PREV, HID)

    est = _estimate(k2, knew, q)
    idx = _topk_pages(est.reshape(H, P))

    att = _sparse_attn(k2, v2, q, knew, vnew, idx)

    out = _proj(att.reshape(1, HID), Wo)
    return out.reshape(1, 1, HID)


# speedup vs baseline: 7.6865x; 7.6865x over previous
import jax
import jax.numpy as jnp
from jax import lax
from jax.experimental import pallas as pl
from jax.experimental.pallas import tpu as pltpu

H = 32
D = 128
HID = 4096
SEQ_PREV = 8191
PAGE = 16
BUDGET = 2048
ROPE_THETA = 10000.0
P = (SEQ_PREV + 1) // PAGE
NSEL = BUDGET // PAGE
SROWS = 256
NSLAB = (SEQ_PREV + 1) // SROWS
NEG = -1e30

INTERPRET = False


def _proj_kernel_rope(x_ref, w_ref, cos_ref, sin_ref, o_ref):
    t = lax.dot_general(x_ref[...], w_ref[...], (((1,), (1,)), ((), ())),
                        preferred_element_type=jnp.float32)
    c = cos_ref[...]
    s = sin_ref[...]
    x1 = t[:, : D // 2]
    x2 = t[:, D // 2:]
    o_ref[0] = jnp.concatenate([x1 * c - x2 * s, x2 * c + x1 * s], axis=1)


def _proj_kernel_plain(x_ref, w_ref, o_ref):
    o_ref[0] = lax.dot_general(x_ref[...], w_ref[...], (((1,), (1,)), ((), ())),
                               preferred_element_type=jnp.float32)


def _proj(x, w, cos=None, sin=None):
    rope = cos is not None
    in_specs = [
        pl.BlockSpec((1, HID), lambda i: (0, 0)),
        pl.BlockSpec((D, HID), lambda i: (i, 0)),
    ]
    args = [x, w]
    if rope:
        in_specs += [pl.BlockSpec((1, D // 2), lambda i: (0, 0)),
                     pl.BlockSpec((1, D // 2), lambda i: (0, 0))]
        args += [cos, sin]
    return pl.pallas_call(
        _proj_kernel_rope if rope else _proj_kernel_plain,
        grid=(HID // D,),
        in_specs=in_specs,
        out_specs=pl.BlockSpec((1, 1, D), lambda i: (i, 0, 0)),
        out_shape=jax.ShapeDtypeStruct((HID // D, 1, D), jnp.float32),
        interpret=INTERPRET,
    )(*args)


def _est_kernel(k_ref, knew_ref, q_ref, est_ref):
    sb = pl.program_id(0)
    blk = k_ref[...]
    row = sb * SROWS + lax.broadcasted_iota(jnp.int32, (SROWS, 1, 1), 0)
    blk = jnp.where(row == SEQ_PREV, knew_ref[...][None], blk)
    pages = blk.reshape(SROWS // PAGE, PAGE, H, D)
    kmin = pages.min(axis=1)
    kmax = pages.max(axis=1)
    q = q_ref[...]
    m = jnp.maximum(q * kmin, q * kmax)
    est_ref[...] = jnp.sum(m, axis=-1, keepdims=True)


def _estimate(k_cache, knew, q):
    return pl.pallas_call(
        _est_kernel,
        grid=(NSLAB,),
        in_specs=[
            pl.BlockSpec((SROWS, H, D), lambda i: (i, 0, 0)),
            pl.BlockSpec((H, D), lambda i: (0, 0)),
            pl.BlockSpec((H, D), lambda i: (0, 0)),
        ],
        out_specs=pl.BlockSpec((SROWS // PAGE, H, 1), lambda i: (i, 0, 0)),
        out_shape=jax.ShapeDtypeStruct((P, H, 1), jnp.float32),
        compiler_params=pltpu.CompilerParams(
            dimension_semantics=("arbitrary",)),
        interpret=INTERPRET,
    )(k_cache, knew, q)


def _cut_kernel(est_ref, bias_ref):
    e = est_ref[...]
    i = lax.bitcast_convert_type(e, jnp.int32)
    enc = jnp.where(i >= 0, i, i ^ jnp.int32(0x7FFFFFFF))
    t = jnp.full((1, H), jnp.int32(-2147483648))
    kk = jnp.int32(NSEL)
    cnt0 = jnp.sum((enc >= 0).astype(jnp.int32), axis=0, keepdims=True)
    t = jnp.where(cnt0 >= kk, jnp.zeros_like(t), t)
    for b in range(30, -1, -1):
        cand = t + jnp.int32(1 << b)
        cnt = jnp.sum((enc >= cand).astype(jnp.int32), axis=0, keepdims=True)
        t = jnp.where(cnt >= kk, cand, t)
    gt = enc > t
    cnt_gt = jnp.sum(gt.astype(jnp.int32), axis=0, keepdims=True)
    need = (kk - cnt_gt).astype(jnp.float32)
    eq = enc == t
    eqf = eq.astype(jnp.float32)
    r = lax.broadcasted_iota(jnp.int32, (P, P), 0)
    c = lax.broadcasted_iota(jnp.int32, (P, P), 1)
    lt = (c <= r).astype(jnp.float32)
    prefix = lax.dot_general(lt, eqf, (((1,), (0,)), ((), ())),
                             preferred_element_type=jnp.float32)
    take = jnp.logical_or(gt, jnp.logical_and(eq, prefix <= need))
    bias_ref[...] = jnp.where(take, 0.0, NEG)


def _page_cut(est):
    return pl.pallas_call(
        _cut_kernel,
        in_specs=[pl.BlockSpec((P, H), lambda: (0, 0))],
        out_specs=pl.BlockSpec((P, H), lambda: (0, 0)),
        out_shape=jax.ShapeDtypeStruct((P, H), jnp.float32),
        interpret=INTERPRET,
    )(est)


def _attn_kernel(k_ref, v_ref, q_ref, knew_ref, vnew_ref, bias_ref, o_ref,
                 acc_ref, m_ref, l_ref):
    i = pl.program_id(0)

    @pl.when(i == 0)
    def _init():
        m_ref[...] = jnp.full((H, 1), NEG)
        l_ref[...] = jnp.zeros((H, 1), jnp.float32)
        acc_ref[...] = jnp.zeros((H, D), jnp.float32)

    kblk = k_ref[...]
    vblk = v_ref[...]
    row = i * SROWS + lax.broadcasted_iota(jnp.int32, (SROWS, 1, 1), 0)
    isnew = row == SEQ_PREV
    kblk = jnp.where(isnew, knew_ref[...][None], kblk)
    vblk = jnp.where(isnew, vnew_ref[...][None], vblk)

    q = q_ref[...]
    s = jnp.sum(kblk * q, axis=-1, keepdims=True)
    s = s * (1.0 / (D ** 0.5)) + bias_ref[...]
    m_prev = m_ref[...]
    m_new = jnp.maximum(m_prev, jnp.max(s, axis=0))
    corr = jnp.exp(m_prev - m_new)
    p = jnp.exp(s - m_new)
    l_ref[...] = l_ref[...] * corr + jnp.sum(p, axis=0)
    acc_ref[...] = acc_ref[...] * corr + jnp.sum(p * vblk, axis=0)
    m_ref[...] = m_new

    @pl.when(i == NSLAB - 1)
    def _fin():
        o_ref[...] = acc_ref[...] / l_ref[...]


def _masked_attn(k_cache, v_cache, q, knew, vnew, bias_rows):
    return pl.pallas_call(
        _attn_kernel,
        grid=(NSLAB,),
        in_specs=[
            pl.BlockSpec((SROWS, H, D), lambda i: (i, 0, 0)),
            pl.BlockSpec((SROWS, H, D), lambda i: (i, 0, 0)),
            pl.BlockSpec((H, D), lambda i: (0, 0)),
            pl.BlockSpec((H, D), lambda i: (0, 0)),
            pl.BlockSpec((H, D), lambda i: (0, 0)),
            pl.BlockSpec((SROWS, H, 1), lambda i: (i, 0, 0)),
        ],
        out_specs=pl.BlockSpec((H, D), lambda i: (0, 0)),
        out_shape=jax.ShapeDtypeStruct((H, D), jnp.float32),
        scratch_shapes=[
            pltpu.VMEM((H, D), jnp.float32),
            pltpu.VMEM((H, 1), jnp.float32),
            pltpu.VMEM((H, 1), jnp.float32),
        ],
        compiler_params=pltpu.CompilerParams(
            dimension_semantics=("arbitrary",)),
        interpret=INTERPRET,
    )(k_cache, v_cache, q, knew, vnew, bias_rows)


def kernel(hidden_states, k_cache, v_cache, Wq, Wk, Wv, Wo):
    x = hidden_states.reshape(1, HID)
    d2 = D // 2
    inv_freq = 1.0 / (ROPE_THETA ** (jnp.arange(0, d2, dtype=jnp.float32) / d2))
    ang = jnp.float32(SEQ_PREV) * inv_freq
    cos = jnp.cos(ang).reshape(1, d2)
    sin = jnp.sin(ang).reshape(1, d2)

    q = _proj(x, Wq, cos, sin).reshape(H, D)
    knew = _proj(x, Wk, cos, sin).reshape(H, D)
    vnew = _proj(x, Wv).reshape(H, D)

    est = _estimate(k_cache, knew, q)
    bias = _page_cut(est.reshape(P, H))
    bias_rows = jnp.repeat(bias, PAGE, axis=0).reshape(SEQ_PREV + 1, H, 1)

    att = _masked_attn(k_cache, v_cache, q, knew, vnew, bias_rows)

    out = _proj(att.reshape(1, HID), Wo)
    return out.reshape(1, 1, HID)
